# baseline (speedup 1.0000x reference)
import jax
import jax.numpy as jnp
from jax.experimental import pallas as pl

B = 4
NPB = 8192
MPB = 2048
C1 = 128
C2 = 256
H = 256
OUT = 256
N = B * NPB

TILE_N = 2048
NT = NPB // TILE_N
TILE_M = 2048
NM = N // TILE_M


def _p1_kernel(u_ref, kt_ref, kf_ref, uf_ref, w1a_ref, w1b_ref,
               y1_ref, s1_ref, q1_ref):
    b = pl.program_id(0)
    t = pl.program_id(1)
    u = u_ref[0]
    kt = kt_ref[0]
    dot = jnp.dot(u, kt, preferred_element_type=jnp.float32,
                  precision=jax.lax.Precision.HIGHEST)
    nu = jnp.sum(u * u, axis=1, keepdims=True)
    nk = jnp.sum(kt * kt, axis=0, keepdims=True)
    d2 = (nu - 2.0 * dot) + nk

    inf = jnp.float32(jnp.inf)
    m1 = jnp.min(d2, axis=1, keepdims=True)
    m2 = jnp.min(jnp.where(d2 == m1, inf, d2), axis=1, keepdims=True)
    m3 = jnp.min(jnp.where(d2 <= m2, inf, d2), axis=1, keepdims=True)
    w1 = 1.0 / (jnp.sqrt(jnp.maximum(m1, 0.0)) + 1e-8)
    w2 = 1.0 / (jnp.sqrt(jnp.maximum(m2, 0.0)) + 1e-8)
    w3 = 1.0 / (jnp.sqrt(jnp.maximum(m3, 0.0)) + 1e-8)
    wsum = w1 + w2 + w3
    S = jnp.where(d2 == m1, w1,
                  jnp.where(d2 == m2, w2,
                            jnp.where(d2 == m3, w3, 0.0)))

    interp = jnp.dot(S, kf_ref[0], preferred_element_type=jnp.float32)
    interp = interp * (1.0 / wsum)
    y1 = (jnp.dot(interp, w1a_ref[...], preferred_element_type=jnp.float32)
          + jnp.dot(uf_ref[0], w1b_ref[...], preferred_element_type=jnp.float32))
    y1_ref[0] = y1

    @pl.when((b == 0) & (t == 0))
    def _init():
        s1_ref[...] = jnp.zeros_like(s1_ref[...])
        q1_ref[...] = jnp.zeros_like(q1_ref[...])

    s1_ref[0:1, :] = s1_ref[0:1, :] + jnp.sum(y1, axis=0, keepdims=True)
    q1_ref[0:1, :] = q1_ref[0:1, :] + jnp.sum(y1 * y1, axis=0, keepdims=True)


def _p2_kernel(y1_ref, s1_ref, q1_ref, g1_ref, b1_ref, w2t_ref,
               y2_ref, s2_ref, q2_ref):
    i = pl.program_id(0)
    mean = s1_ref[0:1, :] * (1.0 / N)
    var = q1_ref[0:1, :] * (1.0 / N) - mean * mean
    rstd = jax.lax.rsqrt(var + 1e-5)
    h = (y1_ref[...] - mean) * (rstd * g1_ref[...]) + b1_ref[...]
    h = jnp.maximum(h, 0.0)
    y2 = jnp.dot(h, w2t_ref[...], preferred_element_type=jnp.float32)
    y2_ref[...] = y2

    @pl.when(i == 0)
    def _init():
        s2_ref[...] = jnp.zeros_like(s2_ref[...])
        q2_ref[...] = jnp.zeros_like(q2_ref[...])

    s2_ref[0:1, :] = s2_ref[0:1, :] + jnp.sum(y2, axis=0, keepdims=True)
    q2_ref[0:1, :] = q2_ref[0:1, :] + jnp.sum(y2 * y2, axis=0, keepdims=True)


def _p3_kernel(y2_ref, s2_ref, q2_ref, g2_ref, b2_ref, o_ref):
    mean = s2_ref[0:1, :] * (1.0 / N)
    var = q2_ref[0:1, :] * (1.0 / N) - mean * mean
    rstd = jax.lax.rsqrt(var + 1e-5)
    o = (y2_ref[...] - mean) * (rstd * g2_ref[...]) + b2_ref[...]
    o_ref[...] = jnp.maximum(o, 0.0)


def kernel(unknown, unknown_batch_cnt, known, known_batch_cnt,
           unknown_feats, known_feats, W1, gamma1, beta1, W2, gamma2, beta2):
    U = jnp.pad(unknown.reshape(B, NPB, 3), ((0, 0), (0, 0), (0, 5)))
    KT = jnp.transpose(known.reshape(B, MPB, 3), (0, 2, 1))
    KT = jnp.pad(KT, ((0, 0), (0, 5), (0, 0)))
    KF = known_feats.reshape(B, MPB, C2)
    UF = unknown_feats.reshape(B, NPB, C1)
    W1T = W1.T
    w1a = W1T[:C2]
    w1b = W1T[C2:]
    W2T = W2.T
    g1 = gamma1.reshape(1, H)
    b1 = beta1.reshape(1, H)
    g2 = gamma2.reshape(1, OUT)
    b2 = beta2.reshape(1, OUT)

    y1, s1, q1 = pl.pallas_call(
        _p1_kernel,
        grid=(B, NT),
        in_specs=[
            pl.BlockSpec((1, TILE_N, 8), lambda b, t: (b, t, 0)),
            pl.BlockSpec((1, 8, MPB), lambda b, t: (b, 0, 0)),
            pl.BlockSpec((1, MPB, C2), lambda b, t: (b, 0, 0)),
            pl.BlockSpec((1, TILE_N, C1), lambda b, t: (b, t, 0)),
            pl.BlockSpec((C2, H), lambda b, t: (0, 0)),
            pl.BlockSpec((C1, H), lambda b, t: (0, 0)),
        ],
        out_specs=[
            pl.BlockSpec((1, TILE_N, H), lambda b, t: (b, t, 0)),
            pl.BlockSpec((8, H), lambda b, t: (0, 0)),
            pl.BlockSpec((8, H), lambda b, t: (0, 0)),
        ],
        out_shape=[
            jax.ShapeDtypeStruct((B, NPB, H), jnp.float32),
            jax.ShapeDtypeStruct((8, H), jnp.float32),
            jax.ShapeDtypeStruct((8, H), jnp.float32),
        ],
    )(U, KT, KF, UF, w1a, w1b)

    y1 = y1.reshape(N, H)
    y2, s2, q2 = pl.pallas_call(
        _p2_kernel,
        grid=(NM,),
        in_specs=[
            pl.BlockSpec((TILE_M, H), lambda i: (i, 0)),
            pl.BlockSpec((8, H), lambda i: (0, 0)),
            pl.BlockSpec((8, H), lambda i: (0, 0)),
            pl.BlockSpec((1, H), lambda i: (0, 0)),
            pl.BlockSpec((1, H), lambda i: (0, 0)),
            pl.BlockSpec((H, OUT), lambda i: (0, 0)),
        ],
        out_specs=[
            pl.BlockSpec((TILE_M, OUT), lambda i: (i, 0)),
            pl.BlockSpec((8, OUT), lambda i: (0, 0)),
            pl.BlockSpec((8, OUT), lambda i: (0, 0)),
        ],
        out_shape=[
            jax.ShapeDtypeStruct((N, OUT), jnp.float32),
            jax.ShapeDtypeStruct((8, OUT), jnp.float32),
            jax.ShapeDtypeStruct((8, OUT), jnp.float32),
        ],
    )(y1, s1, q1, g1, b1, W2T)

    out = pl.pallas_call(
        _p3_kernel,
        grid=(NM,),
        in_specs=[
            pl.BlockSpec((TILE_M, OUT), lambda i: (i, 0)),
            pl.BlockSpec((8, OUT), lambda i: (0, 0)),
            pl.BlockSpec((8, OUT), lambda i: (0, 0)),
            pl.BlockSpec((1, OUT), lambda i: (0, 0)),
            pl.BlockSpec((1, OUT), lambda i: (0, 0)),
        ],
        out_specs=pl.BlockSpec((TILE_M, OUT), lambda i: (i, 0)),
        out_shape=jax.ShapeDtypeStruct((N, OUT), jnp.float32),
    )(y2, s2, q2, g2, b2)

    return out


# speedup vs baseline: 1.5375x; 1.5375x over previous
import jax
import jax.numpy as jnp
from jax.experimental import pallas as pl

B = 4
NPB = 8192
MPB = 2048
C1 = 128
C2 = 256
H = 256
OUT = 256
N = B * NPB

TILE_N = 2048
NT = NPB // TILE_N
TILE_M = 2048
NM = N // TILE_M


def _p1_kernel(u_ref, kt_ref, kf_ref, uf_ref, w1a_ref, w1b_ref,
               y1_ref, s1_ref, q1_ref):
    b = pl.program_id(0)
    t = pl.program_id(1)
    u = u_ref[0]
    kt = kt_ref[0]
    dx = u[:, 0:1] - kt[0:1, :]
    d2 = dx * dx
    dy = u[:, 1:2] - kt[1:2, :]
    d2 = d2 + dy * dy
    dz = u[:, 2:3] - kt[2:3, :]
    d2 = d2 + dz * dz

    inf = jnp.float32(jnp.inf)
    m1 = jnp.min(d2, axis=1, keepdims=True)
    m2 = jnp.min(jnp.where(d2 == m1, inf, d2), axis=1, keepdims=True)
    m3 = jnp.min(jnp.where(d2 <= m2, inf, d2), axis=1, keepdims=True)
    w1 = 1.0 / (jnp.sqrt(jnp.maximum(m1, 0.0)) + 1e-8)
    w2 = 1.0 / (jnp.sqrt(jnp.maximum(m2, 0.0)) + 1e-8)
    w3 = 1.0 / (jnp.sqrt(jnp.maximum(m3, 0.0)) + 1e-8)
    wsum = w1 + w2 + w3
    S = jnp.where(d2 == m1, w1,
                  jnp.where(d2 == m2, w2,
                            jnp.where(d2 == m3, w3, 0.0)))

    interp = jnp.dot(S, kf_ref[0], preferred_element_type=jnp.float32)
    interp = interp * (1.0 / wsum)
    y1 = (jnp.dot(interp, w1a_ref[...], preferred_element_type=jnp.float32)
          + jnp.dot(uf_ref[0], w1b_ref[...], preferred_element_type=jnp.float32))
    y1_ref[0] = y1

    @pl.when((b == 0) & (t == 0))
    def _init():
        s1_ref[...] = jnp.zeros_like(s1_ref[...])
        q1_ref[...] = jnp.zeros_like(q1_ref[...])

    s1_ref[0:1, :] = s1_ref[0:1, :] + jnp.sum(y1, axis=0, keepdims=True)
    q1_ref[0:1, :] = q1_ref[0:1, :] + jnp.sum(y1 * y1, axis=0, keepdims=True)


def _p2_kernel(y1_ref, s1_ref, q1_ref, g1_ref, b1_ref, w2t_ref,
               y2_ref, s2_ref, q2_ref):
    i = pl.program_id(0)
    mean = s1_ref[0:1, :] * (1.0 / N)
    var = q1_ref[0:1, :] * (1.0 / N) - mean * mean
    rstd = jax.lax.rsqrt(var + 1e-5)
    h = (y1_ref[...] - mean) * (rstd * g1_ref[...]) + b1_ref[...]
    h = jnp.maximum(h, 0.0)
    y2 = jnp.dot(h, w2t_ref[...], preferred_element_type=jnp.float32)
    y2_ref[...] = y2

    @pl.when(i == 0)
    def _init():
        s2_ref[...] = jnp.zeros_like(s2_ref[...])
        q2_ref[...] = jnp.zeros_like(q2_ref[...])

    s2_ref[0:1, :] = s2_ref[0:1, :] + jnp.sum(y2, axis=0, keepdims=True)
    q2_ref[0:1, :] = q2_ref[0:1, :] + jnp.sum(y2 * y2, axis=0, keepdims=True)


def _p3_kernel(y2_ref, s2_ref, q2_ref, g2_ref, b2_ref, o_ref):
    mean = s2_ref[0:1, :] * (1.0 / N)
    var = q2_ref[0:1, :] * (1.0 / N) - mean * mean
    rstd = jax.lax.rsqrt(var + 1e-5)
    o = (y2_ref[...] - mean) * (rstd * g2_ref[...]) + b2_ref[...]
    o_ref[...] = jnp.maximum(o, 0.0)


def kernel(unknown, unknown_batch_cnt, known, known_batch_cnt,
           unknown_feats, known_feats, W1, gamma1, beta1, W2, gamma2, beta2):
    U = jnp.pad(unknown.reshape(B, NPB, 3), ((0, 0), (0, 0), (0, 5)))
    KT = jnp.transpose(known.reshape(B, MPB, 3), (0, 2, 1))
    KT = jnp.pad(KT, ((0, 0), (0, 5), (0, 0)))
    KF = known_feats.reshape(B, MPB, C2)
    UF = unknown_feats.reshape(B, NPB, C1)
    W1T = W1.T
    w1a = W1T[:C2]
    w1b = W1T[C2:]
    W2T = W2.T
    g1 = gamma1.reshape(1, H)
    b1 = beta1.reshape(1, H)
    g2 = gamma2.reshape(1, OUT)
    b2 = beta2.reshape(1, OUT)

    y1, s1, q1 = pl.pallas_call(
        _p1_kernel,
        grid=(B, NT),
        in_specs=[
            pl.BlockSpec((1, TILE_N, 8), lambda b, t: (b, t, 0)),
            pl.BlockSpec((1, 8, MPB), lambda b, t: (b, 0, 0)),
            pl.BlockSpec((1, MPB, C2), lambda b, t: (b, 0, 0)),
            pl.BlockSpec((1, TILE_N, C1), lambda b, t: (b, t, 0)),
            pl.BlockSpec((C2, H), lambda b, t: (0, 0)),
            pl.BlockSpec((C1, H), lambda b, t: (0, 0)),
        ],
        out_specs=[
            pl.BlockSpec((1, TILE_N, H), lambda b, t: (b, t, 0)),
            pl.BlockSpec((8, H), lambda b, t: (0, 0)),
            pl.BlockSpec((8, H), lambda b, t: (0, 0)),
        ],
        out_shape=[
            jax.ShapeDtypeStruct((B, NPB, H), jnp.float32),
            jax.ShapeDtypeStruct((8, H), jnp.float32),
            jax.ShapeDtypeStruct((8, H), jnp.float32),
        ],
    )(U, KT, KF, UF, w1a, w1b)

    y1 = y1.reshape(N, H)
    y2, s2, q2 = pl.pallas_call(
        _p2_kernel,
        grid=(NM,),
        in_specs=[
            pl.BlockSpec((TILE_M, H), lambda i: (i, 0)),
            pl.BlockSpec((8, H), lambda i: (0, 0)),
            pl.BlockSpec((8, H), lambda i: (0, 0)),
            pl.BlockSpec((1, H), lambda i: (0, 0)),
            pl.BlockSpec((1, H), lambda i: (0, 0)),
            pl.BlockSpec((H, OUT), lambda i: (0, 0)),
        ],
        out_specs=[
            pl.BlockSpec((TILE_M, OUT), lambda i: (i, 0)),
            pl.BlockSpec((8, OUT), lambda i: (0, 0)),
            pl.BlockSpec((8, OUT), lambda i: (0, 0)),
        ],
        out_shape=[
            jax.ShapeDtypeStruct((N, OUT), jnp.float32),
            jax.ShapeDtypeStruct((8, OUT), jnp.float32),
            jax.ShapeDtypeStruct((8, OUT), jnp.float32),
        ],
    )(y1, s1, q1, g1, b1, W2T)

    out = pl.pallas_call(
        _p3_kernel,
        grid=(NM,),
        in_specs=[
            pl.BlockSpec((TILE_M, OUT), lambda i: (i, 0)),
            pl.BlockSpec((8, OUT), lambda i: (0, 0)),
            pl.BlockSpec((8, OUT), lambda i: (0, 0)),
            pl.BlockSpec((1, OUT), lambda i: (0, 0)),
            pl.BlockSpec((1, OUT), lambda i: (0, 0)),
        ],
        out_specs=pl.BlockSpec((TILE_M, OUT), lambda i: (i, 0)),
        out_shape=jax.ShapeDtypeStruct((N, OUT), jnp.float32),
    )(y2, s2, q2, g2, b2)

    return out


# speedup vs baseline: 1.6003x; 1.0408x over previous
import jax
import jax.numpy as jnp
from jax.experimental import pallas as pl

B = 4
NPB = 8192
MPB = 2048
C1 = 128
C2 = 256
H = 256
OUT = 256
N = B * NPB

TILE_N = 2048
NT = NPB // TILE_N
TILE_M = 2048
NM = N // TILE_M


def _p1_kernel(u_ref, kt_ref, kf_ref, uf_ref, w1a_ref, w1b_ref,
               y1_ref, s1_ref, q1_ref):
    b = pl.program_id(0)
    t = pl.program_id(1)
    u = u_ref[0]
    kt = kt_ref[0]
    dx = u[:, 0:1] - kt[0:1, :]
    d2 = dx * dx
    dy = u[:, 1:2] - kt[1:2, :]
    d2 = d2 + dy * dy
    dz = u[:, 2:3] - kt[2:3, :]
    d2 = d2 + dz * dz

    inf = jnp.float32(jnp.inf)
    m1 = jnp.min(d2, axis=1, keepdims=True)
    m2 = jnp.min(jnp.where(d2 == m1, inf, d2), axis=1, keepdims=True)
    m3 = jnp.min(jnp.where(d2 <= m2, inf, d2), axis=1, keepdims=True)
    w1 = 1.0 / (jnp.sqrt(jnp.maximum(m1, 0.0)) + 1e-8)
    w2 = 1.0 / (jnp.sqrt(jnp.maximum(m2, 0.0)) + 1e-8)
    w3 = 1.0 / (jnp.sqrt(jnp.maximum(m3, 0.0)) + 1e-8)
    wsum = w1 + w2 + w3
    S = jnp.where(d2 == m1, w1,
                  jnp.where(d2 == m2, w2,
                            jnp.where(d2 == m3, w3, 0.0)))

    interp = jnp.dot(S, kf_ref[0], preferred_element_type=jnp.float32)
    interp = interp * (1.0 / wsum)
    y1 = (jnp.dot(interp, w1a_ref[...], preferred_element_type=jnp.float32)
          + jnp.dot(uf_ref[0], w1b_ref[...], preferred_element_type=jnp.float32))
    y1_ref[0] = y1

    @pl.when((b == 0) & (t == 0))
    def _init():
        s1_ref[...] = jnp.zeros_like(s1_ref[...])
        q1_ref[...] = jnp.zeros_like(q1_ref[...])

    s1_ref[0:1, :] = s1_ref[0:1, :] + jnp.sum(y1, axis=0, keepdims=True)
    q1_ref[0:1, :] = q1_ref[0:1, :] + jnp.sum(y1 * y1, axis=0, keepdims=True)


def _p2_kernel(y1_ref, s1_ref, q1_ref, g1_ref, b1_ref, w2t_ref,
               y2_ref, s2_ref, q2_ref):
    i = pl.program_id(0)
    mean = s1_ref[0:1, :] * (1.0 / N)
    var = q1_ref[0:1, :] * (1.0 / N) - mean * mean
    rstd = jax.lax.rsqrt(var + 1e-5)
    h = (y1_ref[...] - mean) * (rstd * g1_ref[...]) + b1_ref[...]
    h = jnp.maximum(h, 0.0)
    y2 = jnp.dot(h, w2t_ref[...], preferred_element_type=jnp.float32)
    y2_ref[...] = y2

    @pl.when(i == 0)
    def _init():
        s2_ref[...] = jnp.zeros_like(s2_ref[...])
        q2_ref[...] = jnp.zeros_like(q2_ref[...])

    s2_ref[0:1, :] = s2_ref[0:1, :] + jnp.sum(y2, axis=0, keepdims=True)
    q2_ref[0:1, :] = q2_ref[0:1, :] + jnp.sum(y2 * y2, axis=0, keepdims=True)


def _p3_kernel(y2_ref, s2_ref, q2_ref, g2_ref, b2_ref, o_ref):
    mean = s2_ref[0:1, :] * (1.0 / N)
    var = q2_ref[0:1, :] * (1.0 / N) - mean * mean
    rstd = jax.lax.rsqrt(var + 1e-5)
    o = (y2_ref[...] - mean) * (rstd * g2_ref[...]) + b2_ref[...]
    o_ref[...] = jnp.maximum(o, 0.0)


def kernel(unknown, unknown_batch_cnt, known, known_batch_cnt,
           unknown_feats, known_feats, W1, gamma1, beta1, W2, gamma2, beta2):
    U = unknown.reshape(B, NPB, 3)
    KT = jnp.transpose(known.reshape(B, MPB, 3), (0, 2, 1))
    KT = jnp.pad(KT, ((0, 0), (0, 5), (0, 0)))
    KF = known_feats.reshape(B, MPB, C2)
    UF = unknown_feats.reshape(B, NPB, C1)
    W1T = W1.T
    w1a = W1T[:C2]
    w1b = W1T[C2:]
    W2T = W2.T
    g1 = gamma1.reshape(1, H)
    b1 = beta1.reshape(1, H)
    g2 = gamma2.reshape(1, OUT)
    b2 = beta2.reshape(1, OUT)

    y1, s1, q1 = pl.pallas_call(
        _p1_kernel,
        grid=(B, NT),
        in_specs=[
            pl.BlockSpec((1, TILE_N, 3), lambda b, t: (b, t, 0)),
            pl.BlockSpec((1, 8, MPB), lambda b, t: (b, 0, 0)),
            pl.BlockSpec((1, MPB, C2), lambda b, t: (b, 0, 0)),
            pl.BlockSpec((1, TILE_N, C1), lambda b, t: (b, t, 0)),
            pl.BlockSpec((C2, H), lambda b, t: (0, 0)),
            pl.BlockSpec((C1, H), lambda b, t: (0, 0)),
        ],
        out_specs=[
            pl.BlockSpec((1, TILE_N, H), lambda b, t: (b, t, 0)),
            pl.BlockSpec((8, H), lambda b, t: (0, 0)),
            pl.BlockSpec((8, H), lambda b, t: (0, 0)),
        ],
        out_shape=[
            jax.ShapeDtypeStruct((B, NPB, H), jnp.float32),
            jax.ShapeDtypeStruct((8, H), jnp.float32),
            jax.ShapeDtypeStruct((8, H), jnp.float32),
        ],
    )(U, KT, KF, UF, w1a, w1b)

    y1 = y1.reshape(N, H)
    y2, s2, q2 = pl.pallas_call(
        _p2_kernel,
        grid=(NM,),
        in_specs=[
            pl.BlockSpec((TILE_M, H), lambda i: (i, 0)),
            pl.BlockSpec((8, H), lambda i: (0, 0)),
            pl.BlockSpec((8, H), lambda i: (0, 0)),
            pl.BlockSpec((1, H), lambda i: (0, 0)),
            pl.BlockSpec((1, H), lambda i: (0, 0)),
            pl.BlockSpec((H, OUT), lambda i: (0, 0)),
        ],
        out_specs=[
            pl.BlockSpec((TILE_M, OUT), lambda i: (i, 0)),
            pl.BlockSpec((8, OUT), lambda i: (0, 0)),
            pl.BlockSpec((8, OUT), lambda i: (0, 0)),
        ],
        out_shape=[
            jax.ShapeDtypeStruct((N, OUT), jnp.float32),
            jax.ShapeDtypeStruct((8, OUT), jnp.float32),
            jax.ShapeDtypeStruct((8, OUT), jnp.float32),
        ],
    )(y1, s1, q1, g1, b1, W2T)

    out = pl.pallas_call(
        _p3_kernel,
        grid=(NM,),
        in_specs=[
            pl.BlockSpec((TILE_M, OUT), lambda i: (i, 0)),
            pl.BlockSpec((8, OUT), lambda i: (0, 0)),
            pl.BlockSpec((8, OUT), lambda i: (0, 0)),
            pl.BlockSpec((1, OUT), lambda i: (0, 0)),
            pl.BlockSpec((1, OUT), lambda i: (0, 0)),
        ],
        out_specs=pl.BlockSpec((TILE_M, OUT), lambda i: (i, 0)),
        out_shape=jax.ShapeDtypeStruct((N, OUT), jnp.float32),
    )(y2, s2, q2, g2, b2)

    return out


# speedup vs baseline: 1.6369x; 1.0228x over previous
import jax
import jax.numpy as jnp
from jax.experimental import pallas as pl

B = 4
NPB = 8192
MPB = 2048
C1 = 128
C2 = 256
H = 256
OUT = 256
N = B * NPB

TILE_N = 2048
NT = NPB // TILE_N
TILE_M = 4096
NM = N // TILE_M


def _p1_kernel(u_ref, kt_ref, kf_ref, uf_ref, w1a_ref, w1b_ref,
               y1_ref, s1_ref, q1_ref):
    b = pl.program_id(0)
    t = pl.program_id(1)
    u = u_ref[0]
    kt = kt_ref[0]
    dx = u[:, 0:1] - kt[0:1, :]
    d2 = dx * dx
    dy = u[:, 1:2] - kt[1:2, :]
    d2 = d2 + dy * dy
    dz = u[:, 2:3] - kt[2:3, :]
    d2 = d2 + dz * dz

    inf = jnp.float32(jnp.inf)
    m1 = jnp.min(d2, axis=1, keepdims=True)
    m2 = jnp.min(jnp.where(d2 == m1, inf, d2), axis=1, keepdims=True)
    m3 = jnp.min(jnp.where(d2 <= m2, inf, d2), axis=1, keepdims=True)
    w1 = 1.0 / (jnp.sqrt(jnp.maximum(m1, 0.0)) + 1e-8)
    w2 = 1.0 / (jnp.sqrt(jnp.maximum(m2, 0.0)) + 1e-8)
    w3 = 1.0 / (jnp.sqrt(jnp.maximum(m3, 0.0)) + 1e-8)
    wsum = w1 + w2 + w3
    S = jnp.where(d2 == m1, w1,
                  jnp.where(d2 == m2, w2,
                            jnp.where(d2 == m3, w3, 0.0)))

    interp = jnp.dot(S, kf_ref[0], preferred_element_type=jnp.float32)
    interp = interp * (1.0 / wsum)
    y1 = (jnp.dot(interp, w1a_ref[...], preferred_element_type=jnp.float32)
          + jnp.dot(uf_ref[0], w1b_ref[...], preferred_element_type=jnp.float32))
    y1_ref[0] = y1

    @pl.when((b == 0) & (t == 0))
    def _init():
        s1_ref[...] = jnp.zeros_like(s1_ref[...])
        q1_ref[...] = jnp.zeros_like(q1_ref[...])

    s1_ref[0:1, :] = s1_ref[0:1, :] + jnp.sum(y1, axis=0, keepdims=True)
    q1_ref[0:1, :] = q1_ref[0:1, :] + jnp.sum(y1 * y1, axis=0, keepdims=True)


def _p2_kernel(y1_ref, s1_ref, q1_ref, g1_ref, b1_ref, w2t_ref,
               y2_ref, s2_ref, q2_ref):
    i = pl.program_id(0)
    mean = s1_ref[0:1, :] * (1.0 / N)
    var = q1_ref[0:1, :] * (1.0 / N) - mean * mean
    rstd = jax.lax.rsqrt(var + 1e-5)
    h = (y1_ref[...] - mean) * (rstd * g1_ref[...]) + b1_ref[...]
    h = jnp.maximum(h, 0.0)
    y2 = jnp.dot(h, w2t_ref[...], preferred_element_type=jnp.float32)
    y2_ref[...] = y2

    @pl.when(i == 0)
    def _init():
        s2_ref[...] = jnp.zeros_like(s2_ref[...])
        q2_ref[...] = jnp.zeros_like(q2_ref[...])

    s2_ref[0:1, :] = s2_ref[0:1, :] + jnp.sum(y2, axis=0, keepdims=True)
    q2_ref[0:1, :] = q2_ref[0:1, :] + jnp.sum(y2 * y2, axis=0, keepdims=True)


def _p3_kernel(y2_ref, s2_ref, q2_ref, g2_ref, b2_ref, o_ref):
    mean = s2_ref[0:1, :] * (1.0 / N)
    var = q2_ref[0:1, :] * (1.0 / N) - mean * mean
    rstd = jax.lax.rsqrt(var + 1e-5)
    o = (y2_ref[...] - mean) * (rstd * g2_ref[...]) + b2_ref[...]
    o_ref[...] = jnp.maximum(o, 0.0)


def kernel(unknown, unknown_batch_cnt, known, known_batch_cnt,
           unknown_feats, known_feats, W1, gamma1, beta1, W2, gamma2, beta2):
    U = unknown.reshape(B, NPB, 3)
    KT = jnp.transpose(known.reshape(B, MPB, 3), (0, 2, 1))
    KT = jnp.pad(KT, ((0, 0), (0, 5), (0, 0)))
    KF = known_feats.reshape(B, MPB, C2)
    UF = unknown_feats.reshape(B, NPB, C1)
    W1T = W1.T
    w1a = W1T[:C2]
    w1b = W1T[C2:]
    W2T = W2.T
    g1 = gamma1.reshape(1, H)
    b1 = beta1.reshape(1, H)
    g2 = gamma2.reshape(1, OUT)
    b2 = beta2.reshape(1, OUT)

    y1, s1, q1 = pl.pallas_call(
        _p1_kernel,
        grid=(B, NT),
        in_specs=[
            pl.BlockSpec((1, TILE_N, 3), lambda b, t: (b, t, 0)),
            pl.BlockSpec((1, 8, MPB), lambda b, t: (b, 0, 0)),
            pl.BlockSpec((1, MPB, C2), lambda b, t: (b, 0, 0)),
            pl.BlockSpec((1, TILE_N, C1), lambda b, t: (b, t, 0)),
            pl.BlockSpec((C2, H), lambda b, t: (0, 0)),
            pl.BlockSpec((C1, H), lambda b, t: (0, 0)),
        ],
        out_specs=[
            pl.BlockSpec((1, TILE_N, H), lambda b, t: (b, t, 0)),
            pl.BlockSpec((8, H), lambda b, t: (0, 0)),
            pl.BlockSpec((8, H), lambda b, t: (0, 0)),
        ],
        out_shape=[
            jax.ShapeDtypeStruct((B, NPB, H), jnp.float32),
            jax.ShapeDtypeStruct((8, H), jnp.float32),
            jax.ShapeDtypeStruct((8, H), jnp.float32),
        ],
    )(U, KT, KF, UF, w1a, w1b)

    y1 = y1.reshape(N, H)
    y2, s2, q2 = pl.pallas_call(
        _p2_kernel,
        grid=(NM,),
        in_specs=[
            pl.BlockSpec((TILE_M, H), lambda i: (i, 0)),
            pl.BlockSpec((8, H), lambda i: (0, 0)),
            pl.BlockSpec((8, H), lambda i: (0, 0)),
            pl.BlockSpec((1, H), lambda i: (0, 0)),
            pl.BlockSpec((1, H), lambda i: (0, 0)),
            pl.BlockSpec((H, OUT), lambda i: (0, 0)),
        ],
        out_specs=[
            pl.BlockSpec((TILE_M, OUT), lambda i: (i, 0)),
            pl.BlockSpec((8, OUT), lambda i: (0, 0)),
            pl.BlockSpec((8, OUT), lambda i: (0, 0)),
        ],
        out_shape=[
            jax.ShapeDtypeStruct((N, OUT), jnp.float32),
            jax.ShapeDtypeStruct((8, OUT), jnp.float32),
            jax.ShapeDtypeStruct((8, OUT), jnp.float32),
        ],
    )(y1, s1, q1, g1, b1, W2T)

    out = pl.pallas_call(
        _p3_kernel,
        grid=(NM,),
        in_specs=[
            pl.BlockSpec((TILE_M, OUT), lambda i: (i, 0)),
            pl.BlockSpec((8, OUT), lambda i: (0, 0)),
            pl.BlockSpec((8, OUT), lambda i: (0, 0)),
            pl.BlockSpec((1, OUT), lambda i: (0, 0)),
            pl.BlockSpec((1, OUT), lambda i: (0, 0)),
        ],
        out_specs=pl.BlockSpec((TILE_M, OUT), lambda i: (i, 0)),
        out_shape=jax.ShapeDtypeStruct((N, OUT), jnp.float32),
    )(y2, s2, q2, g2, b2)

    return out


# speedup vs baseline: 1.6536x; 1.0103x over previous
import jax
import jax.numpy as jnp
from jax.experimental import pallas as pl

B = 4
NPB = 8192
MPB = 2048
C1 = 128
C2 = 256
H = 256
OUT = 256
N = B * NPB

TILE_N = 2048
NT = NPB // TILE_N
TILE_M = 8192
NM = N // TILE_M


def _p1_kernel(u_ref, kt_ref, kf_ref, uf_ref, w1a_ref, w1b_ref,
               y1_ref, s1_ref, q1_ref):
    b = pl.program_id(0)
    t = pl.program_id(1)
    u = u_ref[0]
    kt = kt_ref[0]
    dx = u[:, 0:1] - kt[0:1, :]
    d2 = dx * dx
    dy = u[:, 1:2] - kt[1:2, :]
    d2 = d2 + dy * dy
    dz = u[:, 2:3] - kt[2:3, :]
    d2 = d2 + dz * dz

    inf = jnp.float32(jnp.inf)
    m1 = jnp.min(d2, axis=1, keepdims=True)
    m2 = jnp.min(jnp.where(d2 == m1, inf, d2), axis=1, keepdims=True)
    m3 = jnp.min(jnp.where(d2 <= m2, inf, d2), axis=1, keepdims=True)
    w1 = 1.0 / (jnp.sqrt(jnp.maximum(m1, 0.0)) + 1e-8)
    w2 = 1.0 / (jnp.sqrt(jnp.maximum(m2, 0.0)) + 1e-8)
    w3 = 1.0 / (jnp.sqrt(jnp.maximum(m3, 0.0)) + 1e-8)
    wsum = w1 + w2 + w3
    S = jnp.where(d2 == m1, w1,
                  jnp.where(d2 == m2, w2,
                            jnp.where(d2 == m3, w3, 0.0)))

    interp = jnp.dot(S, kf_ref[0], preferred_element_type=jnp.float32)
    interp = interp * (1.0 / wsum)
    y1 = (jnp.dot(interp, w1a_ref[...], preferred_element_type=jnp.float32)
          + jnp.dot(uf_ref[0], w1b_ref[...], preferred_element_type=jnp.float32))
    y1_ref[0] = y1

    @pl.when((b == 0) & (t == 0))
    def _init():
        s1_ref[...] = jnp.zeros_like(s1_ref[...])
        q1_ref[...] = jnp.zeros_like(q1_ref[...])

    s1_ref[0:1, :] = s1_ref[0:1, :] + jnp.sum(y1, axis=0, keepdims=True)
    q1_ref[0:1, :] = q1_ref[0:1, :] + jnp.sum(y1 * y1, axis=0, keepdims=True)


def _p2_kernel(y1_ref, s1_ref, q1_ref, g1_ref, b1_ref, w2t_ref,
               y2_ref, s2_ref, q2_ref):
    i = pl.program_id(0)
    mean = s1_ref[0:1, :] * (1.0 / N)
    var = q1_ref[0:1, :] * (1.0 / N) - mean * mean
    rstd = jax.lax.rsqrt(var + 1e-5)
    h = (y1_ref[...] - mean) * (rstd * g1_ref[...]) + b1_ref[...]
    h = jnp.maximum(h, 0.0)
    y2 = jnp.dot(h, w2t_ref[...], preferred_element_type=jnp.float32)
    y2_ref[...] = y2

    @pl.when(i == 0)
    def _init():
        s2_ref[...] = jnp.zeros_like(s2_ref[...])
        q2_ref[...] = jnp.zeros_like(q2_ref[...])

    s2_ref[0:1, :] = s2_ref[0:1, :] + jnp.sum(y2, axis=0, keepdims=True)
    q2_ref[0:1, :] = q2_ref[0:1, :] + jnp.sum(y2 * y2, axis=0, keepdims=True)


def _p3_kernel(y2_ref, s2_ref, q2_ref, g2_ref, b2_ref, o_ref):
    mean = s2_ref[0:1, :] * (1.0 / N)
    var = q2_ref[0:1, :] * (1.0 / N) - mean * mean
    rstd = jax.lax.rsqrt(var + 1e-5)
    o = (y2_ref[...] - mean) * (rstd * g2_ref[...]) + b2_ref[...]
    o_ref[...] = jnp.maximum(o, 0.0)


def kernel(unknown, unknown_batch_cnt, known, known_batch_cnt,
           unknown_feats, known_feats, W1, gamma1, beta1, W2, gamma2, beta2):
    U = unknown.reshape(B, NPB, 3)
    KT = jnp.transpose(known.reshape(B, MPB, 3), (0, 2, 1))
    KT = jnp.pad(KT, ((0, 0), (0, 5), (0, 0)))
    KF = known_feats.reshape(B, MPB, C2)
    UF = unknown_feats.reshape(B, NPB, C1)
    W1T = W1.T
    w1a = W1T[:C2]
    w1b = W1T[C2:]
    W2T = W2.T
    g1 = gamma1.reshape(1, H)
    b1 = beta1.reshape(1, H)
    g2 = gamma2.reshape(1, OUT)
    b2 = beta2.reshape(1, OUT)

    y1, s1, q1 = pl.pallas_call(
        _p1_kernel,
        grid=(B, NT),
        in_specs=[
            pl.BlockSpec((1, TILE_N, 3), lambda b, t: (b, t, 0)),
            pl.BlockSpec((1, 8, MPB), lambda b, t: (b, 0, 0)),
            pl.BlockSpec((1, MPB, C2), lambda b, t: (b, 0, 0)),
            pl.BlockSpec((1, TILE_N, C1), lambda b, t: (b, t, 0)),
            pl.BlockSpec((C2, H), lambda b, t: (0, 0)),
            pl.BlockSpec((C1, H), lambda b, t: (0, 0)),
        ],
        out_specs=[
            pl.BlockSpec((1, TILE_N, H), lambda b, t: (b, t, 0)),
            pl.BlockSpec((8, H), lambda b, t: (0, 0)),
            pl.BlockSpec((8, H), lambda b, t: (0, 0)),
        ],
        out_shape=[
            jax.ShapeDtypeStruct((B, NPB, H), jnp.float32),
            jax.ShapeDtypeStruct((8, H), jnp.float32),
            jax.ShapeDtypeStruct((8, H), jnp.float32),
        ],
    )(U, KT, KF, UF, w1a, w1b)

    y1 = y1.reshape(N, H)
    y2, s2, q2 = pl.pallas_call(
        _p2_kernel,
        grid=(NM,),
        in_specs=[
            pl.BlockSpec((TILE_M, H), lambda i: (i, 0)),
            pl.BlockSpec((8, H), lambda i: (0, 0)),
            pl.BlockSpec((8, H), lambda i: (0, 0)),
            pl.BlockSpec((1, H), lambda i: (0, 0)),
            pl.BlockSpec((1, H), lambda i: (0, 0)),
            pl.BlockSpec((H, OUT), lambda i: (0, 0)),
        ],
        out_specs=[
            pl.BlockSpec((TILE_M, OUT), lambda i: (i, 0)),
            pl.BlockSpec((8, OUT), lambda i: (0, 0)),
            pl.BlockSpec((8, OUT), lambda i: (0, 0)),
        ],
        out_shape=[
            jax.ShapeDtypeStruct((N, OUT), jnp.float32),
            jax.ShapeDtypeStruct((8, OUT), jnp.float32),
            jax.ShapeDtypeStruct((8, OUT), jnp.float32),
        ],
    )(y1, s1, q1, g1, b1, W2T)

    out = pl.pallas_call(
        _p3_kernel,
        grid=(NM,),
        in_specs=[
            pl.BlockSpec((TILE_M, OUT), lambda i: (i, 0)),
            pl.BlockSpec((8, OUT), lambda i: (0, 0)),
            pl.BlockSpec((8, OUT), lambda i: (0, 0)),
            pl.BlockSpec((1, OUT), lambda i: (0, 0)),
            pl.BlockSpec((1, OUT), lambda i: (0, 0)),
        ],
        out_specs=pl.BlockSpec((TILE_M, OUT), lambda i: (i, 0)),
        out_shape=jax.ShapeDtypeStruct((N, OUT), jnp.float32),
    )(y2, s2, q2, g2, b2)

    return out
